# baseline (speedup 1.0000x reference)
import functools

import jax
import jax.numpy as jnp
from jax import lax
from jax.experimental import pallas as pl
from jax.experimental.pallas import tpu as pltpu
from jax.experimental.pallas import tpu_sc as plsc

_B, _S = 4, 8192
_NEWLINE = 7
_MAX_ROWS, _MAX_COLS = 100, 200
_D_HALF = 512
_NTOK = _B * _S
_NROWS = 2 * _NTOK
_NC, _NS = 2, 16
_NW = _NC * _NS
_PER_W = _NROWS // _NW
_CHUNK = 32
_NCHUNK = _PER_W // _CHUNK
_NBUF = 2
_DIST = 1
_TROWS = 384
_TPT = _TROWS // _NS


def _pos_body(tok_ref, rows_ref, cols_ref):
    tok = tok_ref[...]
    is_nl = jnp.where(tok == _NEWLINE, 1, 0).astype(jnp.int32)
    pos = lax.broadcasted_iota(jnp.int32, (_B, _S), 1)
    nl_pos = jnp.where(is_nl == 1, pos, -1)

    def shift_right(x, d, fill):
        rolled = pltpu.roll(x, d, axis=1)
        return jnp.where(pos >= d, rolled, fill)

    r = shift_right(is_nl, 1, 0)
    c = shift_right(nl_pos, 1, -1)
    d = 1
    while d < _S:
        r = r + shift_right(r, d, 0)
        c = jnp.maximum(c, shift_right(c, d, -1))
        d *= 2
    rows = jnp.minimum(r, _MAX_ROWS - 1)
    cols = jnp.clip(pos - c - 1, 0, _MAX_COLS - 1) + _MAX_ROWS
    bidx = lax.broadcasted_iota(jnp.int32, (_B, _S), 0)
    repl = ((bidx * _S + pos) >> 10) * _TROWS
    rows_ref[...] = rows + repl
    cols_ref[...] = cols + repl


_positions = pl.pallas_call(
    _pos_body,
    out_shape=(jax.ShapeDtypeStruct((_B, _S), jnp.int32),
               jax.ShapeDtypeStruct((_B, _S), jnp.int32)),
)


@functools.partial(
    pl.kernel,
    out_type=jax.ShapeDtypeStruct((_NROWS, _D_HALF), jnp.float32),
    mesh=plsc.VectorSubcoreMesh(core_axis_name="c", subcore_axis_name="s"),
    compiler_params=pltpu.CompilerParams(needs_layout_passes=False),
    scratch_types=[
        pltpu.VMEM((_PER_W,), jnp.int32),
        pltpu.VMEM((_PER_W,), jnp.int32),
        pltpu.VMEM((_NBUF, _CHUNK, _D_HALF), jnp.float32),
    ]
    + [pltpu.SemaphoreType.DMA] * (2 * _NBUF),
)
def _sc_gather(table_hbm, rows_hbm, cols_hbm, out_hbm, idx_rc, idx_v, bufs, *sems):
    gsem = sems[:_NBUF]
    osem = sems[_NBUF:]
    wid = lax.axis_index("s") * _NC + lax.axis_index("c")
    base = wid * _PER_W
    ntok = _PER_W // 2
    tbase = wid * ntok
    pltpu.sync_copy(rows_hbm.at[pl.ds(tbase, ntok)], idx_rc.at[pl.ds(0, ntok)])
    pltpu.sync_copy(cols_hbm.at[pl.ds(tbase, ntok)], idx_rc.at[pl.ds(ntok, ntok)])
    iota16 = lax.iota(jnp.int32, 16)

    @pl.loop(0, ntok // 16)
    def _mk(k):
        rv = idx_rc[pl.ds(k * 16, 16)]
        cv = idx_rc[pl.ds(ntok + k * 16, 16)]
        pos = k * 32 + 2 * iota16
        plsc.store_scatter(idx_v, [pos], rv)
        plsc.store_scatter(idx_v, [pos + 1], cv)

    def start_g(g, b):
        off = pl.multiple_of(g * _CHUNK, _CHUNK)
        pltpu.async_copy(
            table_hbm.at[idx_v.at[pl.ds(off, _CHUNK)]], bufs.at[b], gsem[b]
        )

    def wait_g(b):
        pltpu.make_async_copy(
            table_hbm.at[pl.ds(0, _CHUNK)], bufs.at[b], gsem[b]
        ).wait()

    def start_o(g, b):
        off = pl.multiple_of(g * _CHUNK, _CHUNK)
        pltpu.async_copy(bufs.at[b], out_hbm.at[pl.ds(base + off, _CHUNK)], osem[b])

    def wait_o(b):
        pltpu.make_async_copy(
            bufs.at[b], out_hbm.at[pl.ds(0, _CHUNK)], osem[b]
        ).wait()

    for j in range(_DIST):
        start_g(j, j)
    for b in range(_NBUF):
        wait_g(b)
        start_o(b, b)
        b2 = (b + _DIST) % _NBUF
        if b >= _DIST:
            wait_o(b2)
        start_g(b + _DIST, b2)

    @pl.loop(1, _NCHUNK // _NBUF - 1)
    def _steady(i):
        g0 = i * _NBUF
        for b in range(_NBUF):
            wait_g(b)
            start_o(g0 + b, b)
            b2 = (b + _DIST) % _NBUF
            wait_o(b2)
            start_g(g0 + b + _DIST, b2)

    gl = _NCHUNK - _NBUF
    for b in range(_NBUF):
        wait_g(b)
        start_o(gl + b, b)
        b2 = (b + _DIST) % _NBUF
        if b < _DIST:
            wait_o(b2)
            start_g(gl + b + _DIST, b2)
    for b in range(_NBUF):
        wait_o(b)


def kernel(token_ids, row_table, col_table):
    rows, cols = _positions(token_ids.astype(jnp.int32))
    pad = jnp.zeros((_TROWS - _MAX_ROWS - _MAX_COLS, _D_HALF), jnp.float32)
    table = jnp.concatenate([row_table, col_table, pad], axis=0)
    table_rep = jnp.tile(table, (_NW, 1))
    out2 = _sc_gather(table_rep, rows.reshape(_NTOK), cols.reshape(_NTOK))
    return out2.reshape(_B, _S, 2 * _D_HALF)


# speedup vs baseline: 1.0622x; 1.0622x over previous
import functools

import jax
import jax.numpy as jnp
from jax import lax
from jax.experimental import pallas as pl
from jax.experimental.pallas import tpu as pltpu
from jax.experimental.pallas import tpu_sc as plsc

_B, _S = 4, 8192
_NEWLINE = 7
_MAX_ROWS, _MAX_COLS = 100, 200
_D_HALF = 512
_NTOK = _B * _S
_NROWS = 2 * _NTOK
_NC, _NS = 2, 16
_NW = _NC * _NS
_PER_W = _NROWS // _NW
_CHUNK = 32
_NCHUNK = _PER_W // _CHUNK
_NBUF = 4
_DIST = 2
_TROWS = 384
_TPT = _TROWS // _NS


def _pos_body(tok_ref, rows_ref, cols_ref):
    tok = tok_ref[...]
    is_nl = jnp.where(tok == _NEWLINE, 1, 0).astype(jnp.int32)
    pos = lax.broadcasted_iota(jnp.int32, (_B, _S), 1)
    nl_pos = jnp.where(is_nl == 1, pos, -1)

    def shift_right(x, d, fill):
        rolled = pltpu.roll(x, d, axis=1)
        return jnp.where(pos >= d, rolled, fill)

    r = shift_right(is_nl, 1, 0)
    c = shift_right(nl_pos, 1, -1)
    d = 1
    while d < _S:
        r = r + shift_right(r, d, 0)
        c = jnp.maximum(c, shift_right(c, d, -1))
        d *= 2
    rows = jnp.minimum(r, _MAX_ROWS - 1)
    cols = jnp.clip(pos - c - 1, 0, _MAX_COLS - 1) + _MAX_ROWS
    bidx = lax.broadcasted_iota(jnp.int32, (_B, _S), 0)
    repl = ((bidx * _S + pos) >> 10) * _TROWS
    rows_ref[...] = rows + repl
    cols_ref[...] = cols + repl


_positions = pl.pallas_call(
    _pos_body,
    out_shape=(jax.ShapeDtypeStruct((_B, _S), jnp.int32),
               jax.ShapeDtypeStruct((_B, _S), jnp.int32)),
)


@functools.partial(
    pl.kernel,
    out_type=jax.ShapeDtypeStruct((_NROWS, _D_HALF), jnp.float32),
    mesh=plsc.VectorSubcoreMesh(core_axis_name="c", subcore_axis_name="s"),
    compiler_params=pltpu.CompilerParams(needs_layout_passes=False),
    scratch_types=[
        pltpu.VMEM((_PER_W,), jnp.int32),
        pltpu.VMEM((_PER_W,), jnp.int32),
        pltpu.VMEM((_NBUF, _CHUNK, _D_HALF), jnp.float32),
    ]
    + [pltpu.SemaphoreType.DMA] * (2 * _NBUF),
)
def _sc_gather(table_hbm, rows_hbm, cols_hbm, out_hbm, idx_rc, idx_v, bufs, *sems):
    gsem = sems[:_NBUF]
    osem = sems[_NBUF:]
    wid = lax.axis_index("s") * _NC + lax.axis_index("c")
    base = wid * _PER_W
    ntok = _PER_W // 2
    tbase = wid * ntok
    pltpu.sync_copy(rows_hbm.at[pl.ds(tbase, ntok)], idx_rc.at[pl.ds(0, ntok)])
    pltpu.sync_copy(cols_hbm.at[pl.ds(tbase, ntok)], idx_rc.at[pl.ds(ntok, ntok)])
    iota16 = lax.iota(jnp.int32, 16)

    @pl.loop(0, ntok // 16)
    def _mk(k):
        rv = idx_rc[pl.ds(k * 16, 16)]
        cv = idx_rc[pl.ds(ntok + k * 16, 16)]
        pos = k * 32 + 2 * iota16
        plsc.store_scatter(idx_v, [pos], rv)
        plsc.store_scatter(idx_v, [pos + 1], cv)

    def start_g(g, b):
        off = pl.multiple_of(g * _CHUNK, _CHUNK)
        pltpu.async_copy(
            table_hbm.at[idx_v.at[pl.ds(off, _CHUNK)]], bufs.at[b], gsem[b]
        )

    def wait_g(b):
        pltpu.make_async_copy(
            table_hbm.at[pl.ds(0, _CHUNK)], bufs.at[b], gsem[b]
        ).wait()

    def start_o(g, b):
        off = pl.multiple_of(g * _CHUNK, _CHUNK)
        pltpu.async_copy(bufs.at[b], out_hbm.at[pl.ds(base + off, _CHUNK)], osem[b])

    def wait_o(b):
        pltpu.make_async_copy(
            bufs.at[b], out_hbm.at[pl.ds(0, _CHUNK)], osem[b]
        ).wait()

    for j in range(_DIST):
        start_g(j, j)
    for b in range(_NBUF):
        wait_g(b)
        start_o(b, b)
        b2 = (b + _DIST) % _NBUF
        if b >= _DIST:
            wait_o(b2)
        start_g(b + _DIST, b2)

    @pl.loop(1, _NCHUNK // _NBUF - 1)
    def _steady(i):
        g0 = i * _NBUF
        for b in range(_NBUF):
            wait_g(b)
            start_o(g0 + b, b)
            b2 = (b + _DIST) % _NBUF
            wait_o(b2)
            start_g(g0 + b + _DIST, b2)

    gl = _NCHUNK - _NBUF
    for b in range(_NBUF):
        wait_g(b)
        start_o(gl + b, b)
        b2 = (b + _DIST) % _NBUF
        if b < _DIST:
            wait_o(b2)
            start_g(gl + b + _DIST, b2)
    for b in range(_NBUF):
        wait_o(b)


def kernel(token_ids, row_table, col_table):
    rows, cols = _positions(token_ids.astype(jnp.int32))
    pad = jnp.zeros((_TROWS - _MAX_ROWS - _MAX_COLS, _D_HALF), jnp.float32)
    table = jnp.concatenate([row_table, col_table, pad], axis=0)
    table_rep = jnp.tile(table, (_NW, 1))
    out2 = _sc_gather(table_rep, rows.reshape(_NTOK), cols.reshape(_NTOK))
    return out2.reshape(_B, _S, 2 * _D_HALF)
